# speedup vs baseline: 2.8074x; 2.8074x over previous
import jax
import jax.numpy as jnp
from jax import lax
from jax.experimental import pallas as pl
from jax.experimental.pallas import tpu as pltpu
from jax.experimental.pallas import tpu_sc as plsc

N_MESHES = 16
V_PER = 2048
F_PER = 4096
D_FEAT = 128

NC = 2
NS = 16
NW = NC * NS

R_TOTAL = N_MESHES * F_PER * 3
R_PER_W = R_TOTAL // NW
CHUNK = 128
NCHUNK = R_PER_W // CHUNK
IDX_ROWS = R_PER_W // 128


def _gather_body(fidx_hbm, feats_hbm, out_hbm, idx_v, rows_v, sem):
    wid = lax.axis_index("s") * NC + lax.axis_index("c")
    base = wid * R_PER_W
    pltpu.sync_copy(fidx_hbm.at[pl.ds(wid * IDX_ROWS, IDX_ROWS)], idx_v)
    offv = jnp.full((16,), (wid // 2) * V_PER, dtype=jnp.int32)

    def _add(t, carry):
        c = t // 8
        j = (t % 8) * 16
        idx_v[c, pl.ds(j, 16)] = idx_v[c, pl.ds(j, 16)] + offv
        return carry

    lax.fori_loop(0, IDX_ROWS * 8, _add, 0)

    def _chunk(c, carry):
        pltpu.async_copy(feats_hbm.at[idx_v.at[c]], rows_v, sem).wait()
        pltpu.sync_copy(rows_v, out_hbm.at[pl.ds(base + c * CHUNK, CHUNK)])
        return carry

    lax.fori_loop(0, NCHUNK, _chunk, 0)


@jax.jit
def _gather_rows(fidx2d, feats):
    mesh = plsc.VectorSubcoreMesh(core_axis_name="c", subcore_axis_name="s")
    return pl.kernel(
        _gather_body,
        mesh=mesh,
        out_type=jax.ShapeDtypeStruct((R_TOTAL, D_FEAT), jnp.float32),
        scratch_types=[
            pltpu.VMEM((IDX_ROWS, 128), jnp.int32),
            pltpu.VMEM((CHUNK, D_FEAT), jnp.float32),
            pltpu.SemaphoreType.DMA,
        ],
    )(fidx2d, feats)


def _ranges_body(v_ref, out_ref):
    i = pl.program_id(0)
    v = v_ref[...]
    r = jnp.max(v, axis=0) - jnp.min(v, axis=0)
    out_ref[pl.ds(i, 1), :] = r[None, :]


@jax.jit
def _ranges(verts):
    return pl.pallas_call(
        _ranges_body,
        grid=(N_MESHES,),
        in_specs=[pl.BlockSpec((V_PER, 3), lambda i: (i, 0))],
        out_specs=pl.BlockSpec((N_MESHES, 3), lambda i: (0, 0)),
        out_shape=jax.ShapeDtypeStruct((N_MESHES, 3), jnp.float32),
    )(verts)


def kernel(verts, faces, feats):
    fidx2d = faces.reshape(R_TOTAL // 128, 128)
    rows = _gather_rows(fidx2d, feats)
    feats_from_faces = rows.reshape(N_MESHES * F_PER, 3, D_FEAT)
    ranges = _ranges(verts)
    return feats_from_faces, ranges


# speedup vs baseline: 2.9221x; 1.0408x over previous
import jax
import jax.numpy as jnp
from jax import lax
from jax.experimental import pallas as pl
from jax.experimental.pallas import tpu as pltpu
from jax.experimental.pallas import tpu_sc as plsc

N_MESHES = 16
V_PER = 2048
F_PER = 4096
D_FEAT = 128

NC = 2
NS = 16
NW = NC * NS

R_TOTAL = N_MESHES * F_PER * 3
R_PER_W = R_TOTAL // NW
CHUNK = 128
NCHUNK = R_PER_W // CHUNK
IDX_ROWS = R_PER_W // 128


def _gather_body(fidx_hbm, feats_hbm, out_hbm, idx_v, rows_v, sem0, sem1):
    wid = lax.axis_index("s") * NC + lax.axis_index("c")
    base = wid * R_PER_W
    pltpu.sync_copy(fidx_hbm.at[pl.ds(wid * IDX_ROWS, IDX_ROWS)], idx_v)
    offv = jnp.full((16,), (wid // 2) * V_PER, dtype=jnp.int32)

    def _add(t, carry):
        c = t // 8
        j = (t % 8) * 16
        idx_v[c, pl.ds(j, 16)] = idx_v[c, pl.ds(j, 16)] + offv
        return carry

    lax.fori_loop(0, IDX_ROWS * 8, _add, 0)

    sems = (sem0, sem1)
    bufs = (rows_v.at[0], rows_v.at[1])

    pltpu.async_copy(feats_hbm.at[idx_v.at[0]], bufs[0], sems[0])

    def _outer(g, carry):
        for b in (0, 1):
            c = 2 * g + b
            nb = 1 - b
            pltpu.make_async_copy(feats_hbm.at[idx_v.at[c]], bufs[b], sems[b]).wait()

            @pl.when(c + 1 < NCHUNK)
            def _():
                pltpu.async_copy(feats_hbm.at[idx_v.at[c + 1]], bufs[nb], sems[nb])

            pltpu.sync_copy(bufs[b], out_hbm.at[pl.ds(base + c * CHUNK, CHUNK)])
        return carry

    lax.fori_loop(0, NCHUNK // 2, _outer, 0)


@jax.jit
def _gather_rows(fidx2d, feats):
    mesh = plsc.VectorSubcoreMesh(core_axis_name="c", subcore_axis_name="s")
    return pl.kernel(
        _gather_body,
        mesh=mesh,
        out_type=jax.ShapeDtypeStruct((R_TOTAL, D_FEAT), jnp.float32),
        scratch_types=[
            pltpu.VMEM((IDX_ROWS, 128), jnp.int32),
            pltpu.VMEM((2, CHUNK, D_FEAT), jnp.float32),
            pltpu.SemaphoreType.DMA,
            pltpu.SemaphoreType.DMA,
        ],
    )(fidx2d, feats)


def _ranges_body(v_ref, out_ref):
    i = pl.program_id(0)
    v = v_ref[...]
    r = jnp.max(v, axis=0) - jnp.min(v, axis=0)
    out_ref[pl.ds(i, 1), :] = r[None, :]


@jax.jit
def _ranges(verts):
    return pl.pallas_call(
        _ranges_body,
        grid=(N_MESHES,),
        in_specs=[pl.BlockSpec((V_PER, 3), lambda i: (i, 0))],
        out_specs=pl.BlockSpec((N_MESHES, 3), lambda i: (0, 0)),
        out_shape=jax.ShapeDtypeStruct((N_MESHES, 3), jnp.float32),
    )(verts)


def kernel(verts, faces, feats):
    fidx2d = faces.reshape(R_TOTAL // 128, 128)
    rows = _gather_rows(fidx2d, feats)
    feats_from_faces = rows.reshape(N_MESHES * F_PER, 3, D_FEAT)
    ranges = _ranges(verts)
    return feats_from_faces, ranges


# speedup vs baseline: 7.1741x; 2.4551x over previous
import jax
import jax.numpy as jnp
from jax import lax
from jax.experimental import pallas as pl
from jax.experimental.pallas import tpu as pltpu
from jax.experimental.pallas import tpu_sc as plsc

N_MESHES = 16
V_PER = 2048
F_PER = 4096
D_FEAT = 128

NC = 2
NS = 16
NW = NC * NS

F_TOTAL = N_MESHES * F_PER
R_TOTAL = F_TOTAL * 3
R_PER_W = R_TOTAL // NW
CHUNK = 128
NCHUNK = R_PER_W // CHUNK
IDX_ROWS = R_PER_W // 128


def _gather_body(fidx_hbm, feats_hbm, out_hbm, idx_v, oidx_v, rows_v,
                 gsem0, gsem1, ssem0, ssem1):
    wid = lax.axis_index("s") * NC + lax.axis_index("c")
    base = wid * R_PER_W
    pltpu.sync_copy(fidx_hbm.at[pl.ds(wid * IDX_ROWS, IDX_ROWS)], idx_v)
    offv = jnp.full((16,), (wid // 2) * V_PER, dtype=jnp.int32)

    def _iv(val):
        return jnp.full((16,), val, dtype=jnp.int32)

    lanes = lax.iota(jnp.int32, 16)
    m = lanes
    dsm = _iv(0)
    for _ in range(5):
        ge = m >= _iv(3)
        m = jnp.where(ge, m - _iv(3), m)
        dsm = jnp.where(ge, dsm + _iv(1), dsm)
    d0 = dsm + _iv(wid * (R_PER_W // 3))

    def _prep(t, carry):
        d, m = carry
        c = t // 8
        j = (t % 8) * 16
        idx_v[c, pl.ds(j, 16)] = idx_v[c, pl.ds(j, 16)] + offv
        oidx_v[c, pl.ds(j, 16)] = m * _iv(F_TOTAL) + d
        is2 = m == _iv(2)
        d = d + jnp.where(is2, _iv(6), _iv(5))
        m = jnp.where(is2, _iv(0), m + _iv(1))
        return (d, m)

    lax.fori_loop(0, IDX_ROWS * 8, _prep, (d0, m))

    gsems = (gsem0, gsem1)
    ssems = (ssem0, ssem1)
    bufs = (rows_v.at[0], rows_v.at[1])

    pltpu.async_copy(feats_hbm.at[idx_v.at[0]], bufs[0], gsems[0])

    def _outer(g, carry):
        for b in (0, 1):
            c = 2 * g + b
            nb = 1 - b
            pltpu.make_async_copy(feats_hbm.at[idx_v.at[c]], bufs[b],
                                  gsems[b]).wait()

            @pl.when(c >= 1)
            def _():
                pltpu.make_async_copy(bufs[nb], out_hbm.at[oidx_v.at[c - 1]],
                                      ssems[nb]).wait()

            @pl.when(c + 1 < NCHUNK)
            def _():
                pltpu.async_copy(feats_hbm.at[idx_v.at[c + 1]], bufs[nb],
                                 gsems[nb])

            pltpu.async_copy(bufs[b], out_hbm.at[oidx_v.at[c]], ssems[b])
        return carry

    lax.fori_loop(0, NCHUNK // 2, _outer, 0)
    lb = (NCHUNK - 1) % 2
    pltpu.make_async_copy(bufs[lb], out_hbm.at[oidx_v.at[NCHUNK - 1]],
                          ssems[lb]).wait()


@jax.jit
def _gather_rows(fidx2d, feats):
    mesh = plsc.VectorSubcoreMesh(core_axis_name="c", subcore_axis_name="s")
    return pl.kernel(
        _gather_body,
        mesh=mesh,
        out_type=jax.ShapeDtypeStruct((R_TOTAL, D_FEAT), jnp.float32),
        scratch_types=[
            pltpu.VMEM((IDX_ROWS, 128), jnp.int32),
            pltpu.VMEM((IDX_ROWS, 128), jnp.int32),
            pltpu.VMEM((2, CHUNK, D_FEAT), jnp.float32),
            pltpu.SemaphoreType.DMA,
            pltpu.SemaphoreType.DMA,
            pltpu.SemaphoreType.DMA,
            pltpu.SemaphoreType.DMA,
        ],
    )(fidx2d, feats)


def _ranges_body(v_ref, out_ref):
    i = pl.program_id(0)
    v = v_ref[...]
    r = jnp.max(v, axis=0) - jnp.min(v, axis=0)
    out_ref[pl.ds(i, 1), :] = r[None, :]


@jax.jit
def _ranges(verts):
    return pl.pallas_call(
        _ranges_body,
        grid=(N_MESHES,),
        in_specs=[pl.BlockSpec((V_PER, 3), lambda i: (i, 0))],
        out_specs=pl.BlockSpec((N_MESHES, 3), lambda i: (0, 0)),
        out_shape=jax.ShapeDtypeStruct((N_MESHES, 3), jnp.float32),
    )(verts)


def kernel(verts, faces, feats):
    fidx2d = faces.reshape(R_TOTAL // 128, 128)
    rows = _gather_rows(fidx2d, feats)
    feats_from_faces = rows.reshape(3, F_TOTAL, D_FEAT).transpose(1, 0, 2)
    ranges = _ranges(verts)
    return feats_from_faces, ranges


# speedup vs baseline: 11.1236x; 1.5505x over previous
import jax
import jax.numpy as jnp
from jax import lax
from jax.experimental import pallas as pl
from jax.experimental.pallas import tpu as pltpu
from jax.experimental.pallas import tpu_sc as plsc

N_MESHES = 16
V_PER = 2048
F_PER = 4096
D_FEAT = 128

NC = 2
NS = 16
NW = NC * NS

F_TOTAL = N_MESHES * F_PER
R_TOTAL = F_TOTAL * 3
R_PER_W = R_TOTAL // NW
CHUNK = 128
NCHUNK = R_PER_W // CHUNK
NBUF = 4
IDX_ROWS = R_PER_W // 128
CHUNKS_PER_MESH_COL = (F_PER * 3) // CHUNK


def _gather_body(fidx_hbm, feats_hbm, out_hbm, idx_v, rows_v,
                 gsem0, gsem1, gsem2, gsem3, wsem0, wsem1, wsem2, wsem3):
    wid = lax.axis_index("s") * NC + lax.axis_index("c")
    base = wid * R_PER_W
    pltpu.sync_copy(fidx_hbm.at[pl.ds(base, R_PER_W)], idx_v)

    def _prep(t, carry):
        k = wid * IDX_ROWS + t // 8
        mesh = (k % (F_TOTAL // CHUNK)) // (F_PER // CHUNK)
        offv = jnp.full((16,), mesh * V_PER, dtype=jnp.int32)
        idx_v[pl.ds(t * 16, 16)] = idx_v[pl.ds(t * 16, 16)] + offv
        return carry

    lax.fori_loop(0, IDX_ROWS * 8, _prep, 0)

    gsems = (gsem0, gsem1, gsem2, gsem3)
    wsems = (wsem0, wsem1, wsem2, wsem3)
    bufs = tuple(rows_v.at[i] for i in range(NBUF))

    def _gather(c, b):
        return pltpu.make_async_copy(
            feats_hbm.at[idx_v.at[pl.ds(c * CHUNK, CHUNK)]], bufs[b], gsems[b])

    def _write(c, b):
        return pltpu.make_async_copy(
            bufs[b], out_hbm.at[pl.ds(base + c * CHUNK, CHUNK)], wsems[b])

    _gather(0, 0).start()
    _gather(1, 1).start()

    def _outer(g, carry):
        for b in range(NBUF):
            c = NBUF * g + b
            _gather(c, b).wait()
            _write(c, b).start()

            nb = (b + 2) % NBUF

            @pl.when(c >= 2)
            def _():
                _write(c - 2, nb).wait()

            @pl.when(c + 2 < NCHUNK)
            def _():
                _gather(c + 2, nb).start()
        return carry

    lax.fori_loop(0, NCHUNK // NBUF, _outer, 0)
    _write(NCHUNK - 2, (NCHUNK - 2) % NBUF).wait()
    _write(NCHUNK - 1, (NCHUNK - 1) % NBUF).wait()


@jax.jit
def _gather_rows(fidx, feats):
    mesh = plsc.VectorSubcoreMesh(core_axis_name="c", subcore_axis_name="s")
    return pl.kernel(
        _gather_body,
        mesh=mesh,
        out_type=jax.ShapeDtypeStruct((R_TOTAL, D_FEAT), jnp.float32),
        scratch_types=[
            pltpu.VMEM((R_PER_W,), jnp.int32),
            pltpu.VMEM((NBUF, CHUNK, D_FEAT), jnp.float32),
            pltpu.SemaphoreType.DMA,
            pltpu.SemaphoreType.DMA,
            pltpu.SemaphoreType.DMA,
            pltpu.SemaphoreType.DMA,
            pltpu.SemaphoreType.DMA,
            pltpu.SemaphoreType.DMA,
            pltpu.SemaphoreType.DMA,
            pltpu.SemaphoreType.DMA,
        ],
    )(fidx, feats)


def _ranges_body(v_ref, out_ref):
    i = pl.program_id(0)
    v = v_ref[...]
    r = jnp.max(v, axis=0) - jnp.min(v, axis=0)
    out_ref[pl.ds(i, 1), :] = r[None, :]


@jax.jit
def _ranges(verts):
    return pl.pallas_call(
        _ranges_body,
        grid=(N_MESHES,),
        in_specs=[pl.BlockSpec((V_PER, 3), lambda i: (i, 0))],
        out_specs=pl.BlockSpec((N_MESHES, 3), lambda i: (0, 0)),
        out_shape=jax.ShapeDtypeStruct((N_MESHES, 3), jnp.float32),
    )(verts)


def kernel(verts, faces, feats):
    fidx = faces.T.reshape(-1)
    rows = _gather_rows(fidx, feats)
    feats_from_faces = rows.reshape(3, F_TOTAL, D_FEAT).transpose(1, 0, 2)
    ranges = _ranges(verts)
    return feats_from_faces, ranges


# speedup vs baseline: 11.2912x; 1.0151x over previous
import jax
import jax.numpy as jnp
from jax import lax
from jax.experimental import pallas as pl
from jax.experimental.pallas import tpu as pltpu
from jax.experimental.pallas import tpu_sc as plsc

N_MESHES = 16
V_PER = 2048
F_PER = 4096
D_FEAT = 128

NC = 2
NS = 16
NW = NC * NS

F_TOTAL = N_MESHES * F_PER
R_TOTAL = F_TOTAL * 3
R_PER_W = R_TOTAL // NW
CHUNK = 128
NCHUNK = R_PER_W // CHUNK
NBUF = 6
LOOKAHEAD = NBUF // 2
IDX_ROWS = R_PER_W // 128
CHUNKS_PER_MESH_COL = (F_PER * 3) // CHUNK


def _gather_body(fidx_hbm, feats_hbm, out_hbm, idx_v, rows_v,
                 gsem0, gsem1, gsem2, gsem3, gsem4, gsem5,
                 wsem0, wsem1, wsem2, wsem3, wsem4, wsem5):
    wid = lax.axis_index("s") * NC + lax.axis_index("c")
    base = wid * R_PER_W
    pltpu.sync_copy(fidx_hbm.at[pl.ds(base, R_PER_W)], idx_v)

    def _prep(t, carry):
        k = wid * IDX_ROWS + t // 8
        mesh = (k % (F_TOTAL // CHUNK)) // (F_PER // CHUNK)
        offv = jnp.full((16,), mesh * V_PER, dtype=jnp.int32)
        idx_v[pl.ds(t * 16, 16)] = idx_v[pl.ds(t * 16, 16)] + offv
        return carry

    lax.fori_loop(0, IDX_ROWS * 8, _prep, 0)

    gsems = (gsem0, gsem1, gsem2, gsem3, gsem4, gsem5)
    wsems = (wsem0, wsem1, wsem2, wsem3, wsem4, wsem5)
    bufs = tuple(rows_v.at[i] for i in range(NBUF))

    def _gather(c, b):
        return pltpu.make_async_copy(
            feats_hbm.at[idx_v.at[pl.ds(c * CHUNK, CHUNK)]], bufs[b], gsems[b])

    def _write(c, b):
        return pltpu.make_async_copy(
            bufs[b], out_hbm.at[pl.ds(base + c * CHUNK, CHUNK)], wsems[b])

    for i in range(LOOKAHEAD):
        _gather(i, i).start()

    def _outer(g, carry):
        for b in range(NBUF):
            c = NBUF * g + b
            _gather(c, b).wait()
            _write(c, b).start()

            nb = (b + LOOKAHEAD) % NBUF

            @pl.when(c >= LOOKAHEAD)
            def _():
                _write(c - LOOKAHEAD, nb).wait()

            @pl.when(c + LOOKAHEAD < NCHUNK)
            def _():
                _gather(c + LOOKAHEAD, nb).start()
        return carry

    lax.fori_loop(0, NCHUNK // NBUF, _outer, 0)
    for c in range(NCHUNK - LOOKAHEAD, NCHUNK):
        _write(c, c % NBUF).wait()


@jax.jit
def _gather_rows(fidx, feats):
    mesh = plsc.VectorSubcoreMesh(core_axis_name="c", subcore_axis_name="s")
    return pl.kernel(
        _gather_body,
        mesh=mesh,
        out_type=jax.ShapeDtypeStruct((R_TOTAL, D_FEAT), jnp.float32),
        scratch_types=[
            pltpu.VMEM((R_PER_W,), jnp.int32),
            pltpu.VMEM((NBUF, CHUNK, D_FEAT), jnp.float32),
        ] + [pltpu.SemaphoreType.DMA] * (2 * NBUF),
    )(fidx, feats)


def _ranges_body(v_ref, out_ref):
    i = pl.program_id(0)
    v = v_ref[...]
    r = jnp.max(v, axis=0) - jnp.min(v, axis=0)
    out_ref[pl.ds(i, 1), :] = r[None, :]


@jax.jit
def _ranges(verts):
    return pl.pallas_call(
        _ranges_body,
        grid=(N_MESHES,),
        in_specs=[pl.BlockSpec((V_PER, 3), lambda i: (i, 0))],
        out_specs=pl.BlockSpec((N_MESHES, 3), lambda i: (0, 0)),
        out_shape=jax.ShapeDtypeStruct((N_MESHES, 3), jnp.float32),
    )(verts)


def kernel(verts, faces, feats):
    fidx = faces.T.reshape(-1)
    rows = _gather_rows(fidx, feats)
    feats_from_faces = rows.reshape(3, F_TOTAL, D_FEAT).transpose(1, 0, 2)
    ranges = _ranges(verts)
    return feats_from_faces, ranges


# speedup vs baseline: 12.6022x; 1.1161x over previous
import jax
import jax.numpy as jnp
from jax import lax
from jax.experimental import pallas as pl
from jax.experimental.pallas import tpu as pltpu
from jax.experimental.pallas import tpu_sc as plsc

N_MESHES = 16
V_PER = 2048
F_PER = 4096
D_FEAT = 128

NC = 2
NS = 16

F_TOTAL = N_MESHES * F_PER
R_TOTAL = F_TOTAL * 3
CHUNK = 128
NQ = F_TOTAL // CHUNK
NBUF = 4
LOOKAHEAD = NBUF // 2
PHASES = 4
SLABS = 2
SLAB_ROWS = SLABS * V_PER
CPW = 4
CPP = 3 * CPW
IDX_PP = CPP * CHUNK


def _gather_body(fidx_hbm, feats_hbm, out_hbm, idx_v, rows_v, slab_sh, *sems):
    gsems = sems[:NBUF]
    wsems = sems[NBUF:2 * NBUF]
    core = lax.axis_index("c")
    sub = lax.axis_index("s")
    bufs = tuple(rows_v.at[i] for i in range(NBUF))

    for t in range(PHASES):
        slab_row0 = (core * 8 + t * SLABS) * V_PER
        pltpu.sync_copy(
            feats_hbm.at[pl.ds(slab_row0 + sub * (SLAB_ROWS // NS),
                               SLAB_ROWS // NS)],
            slab_sh.at[pl.ds(sub * (SLAB_ROWS // NS), SLAB_ROWS // NS)])
        plsc.subcore_barrier()

        qbase = core * (PHASES * NS * CPW) + t * (NS * CPW) + sub * CPW

        for p in range(3):
            pltpu.sync_copy(
                fidx_hbm.at[pl.ds((p * NQ + qbase) * CHUNK, CPW * CHUNK)],
                idx_v.at[pl.ds(p * CPW * CHUNK, CPW * CHUNK)])

        offv = jnp.full((16,), (sub // 8) * V_PER, dtype=jnp.int32)

        def _prep(i, carry):
            idx_v[pl.ds(i * 16, 16)] = idx_v[pl.ds(i * 16, 16)] + offv
            return carry

        lax.fori_loop(0, IDX_PP // 16, _prep, 0)

        def _gather(i, b):
            return pltpu.make_async_copy(
                slab_sh.at[idx_v.at[pl.ds(i * CHUNK, CHUNK)]],
                bufs[b], gsems[b])

        def _write(i, b):
            p = i // CPW
            k = p * NQ + qbase + (i % CPW)
            return pltpu.make_async_copy(
                bufs[b], out_hbm.at[pl.ds(k * CHUNK, CHUNK)], wsems[b])

        for i in range(LOOKAHEAD):
            _gather(i, i).start()

        def _outer(g, carry):
            for b in range(NBUF):
                i = NBUF * g + b
                _gather(i, b).wait()
                _write(i, b).start()

                nb = (b + LOOKAHEAD) % NBUF

                @pl.when(i >= LOOKAHEAD)
                def _():
                    _write(i - LOOKAHEAD, nb).wait()

                @pl.when(i + LOOKAHEAD < CPP)
                def _():
                    _gather(i + LOOKAHEAD, nb).start()
            return carry

        lax.fori_loop(0, CPP // NBUF, _outer, 0)
        for i in range(CPP - LOOKAHEAD, CPP):
            _write(i, i % NBUF).wait()
        plsc.subcore_barrier()


@jax.jit
def _gather_rows(fidx, feats):
    mesh = plsc.VectorSubcoreMesh(core_axis_name="c", subcore_axis_name="s")
    return pl.kernel(
        _gather_body,
        mesh=mesh,
        out_type=jax.ShapeDtypeStruct((R_TOTAL, D_FEAT), jnp.float32),
        scratch_types=[
            pltpu.VMEM((IDX_PP,), jnp.int32),
            pltpu.VMEM((NBUF, CHUNK, D_FEAT), jnp.float32),
            pltpu.VMEM_SHARED((SLAB_ROWS, D_FEAT), jnp.float32),
        ] + [pltpu.SemaphoreType.DMA] * (2 * NBUF),
    )(fidx, feats)


def _ranges_body(v_ref, out_ref):
    i = pl.program_id(0)
    v = v_ref[...]
    r = jnp.max(v, axis=0) - jnp.min(v, axis=0)
    out_ref[pl.ds(i, 1), :] = r[None, :]


@jax.jit
def _ranges(verts):
    return pl.pallas_call(
        _ranges_body,
        grid=(N_MESHES,),
        in_specs=[pl.BlockSpec((V_PER, 3), lambda i: (i, 0))],
        out_specs=pl.BlockSpec((N_MESHES, 3), lambda i: (0, 0)),
        out_shape=jax.ShapeDtypeStruct((N_MESHES, 3), jnp.float32),
    )(verts)


def kernel(verts, faces, feats):
    fidx = faces.T.reshape(-1)
    rows = _gather_rows(fidx, feats)
    feats_from_faces = rows.reshape(3, F_TOTAL, D_FEAT).transpose(1, 0, 2)
    ranges = _ranges(verts)
    return feats_from_faces, ranges
